# baseline (speedup 1.0000x reference)
import functools

import jax
import jax.numpy as jnp
from jax import lax
from jax.experimental import pallas as pl
from jax.experimental.pallas import tpu as pltpu
from jax.experimental.pallas import tpu_sc as plsc

_NC = 2
_NS = 16
_NW = _NC * _NS


def _sc_gather_body(bpw, users_hbm, items_hbm, embu_hbm, embi_hbm, th_hbm,
                    ph_hbm, urows_out, irows_out, th_out, ph_out,
                    uidx_v, iidx_v, urows_v, irows_v, th_v, ph_v, sem):
    wid = lax.axis_index("s") * _NC + lax.axis_index("c")
    base = wid * bpw
    pltpu.sync_copy(users_hbm.at[pl.ds(base, bpw)], uidx_v)
    pltpu.sync_copy(items_hbm.at[pl.ds(base, bpw)], iidx_v)
    c1 = pltpu.async_copy(embu_hbm.at[uidx_v], urows_v, sem)
    c2 = pltpu.async_copy(embi_hbm.at[iidx_v], irows_v, sem)
    c3 = pltpu.async_copy(th_hbm.at[uidx_v], th_v, sem)
    c4 = pltpu.async_copy(ph_hbm.at[iidx_v], ph_v, sem)
    c1.wait()
    c2.wait()
    c3.wait()
    c4.wait()
    pltpu.sync_copy(urows_v, urows_out.at[pl.ds(base, bpw)])
    pltpu.sync_copy(irows_v, irows_out.at[pl.ds(base, bpw)])
    pltpu.sync_copy(th_v, th_out.at[pl.ds(base, bpw)])
    pltpu.sync_copy(ph_v, ph_out.at[pl.ds(base, bpw)])


@functools.lru_cache(maxsize=None)
def _make_sc_gather(B, D):
    assert B % (8 * _NW) == 0
    bpw = B // _NW
    mesh = plsc.VectorSubcoreMesh(core_axis_name="c", subcore_axis_name="s")
    f32 = jnp.float32
    return pl.kernel(
        functools.partial(_sc_gather_body, bpw),
        mesh=mesh,
        out_type=[
            jax.ShapeDtypeStruct((B, D), f32),
            jax.ShapeDtypeStruct((B, D), f32),
            jax.ShapeDtypeStruct((B,), f32),
            jax.ShapeDtypeStruct((B,), f32),
        ],
        scratch_types=[
            pltpu.VMEM((bpw,), jnp.int32),
            pltpu.VMEM((bpw,), jnp.int32),
            pltpu.VMEM((bpw, D), f32),
            pltpu.VMEM((bpw, D), f32),
            pltpu.VMEM((bpw,), f32),
            pltpu.VMEM((bpw,), f32),
            pltpu.SemaphoreType.DMA,
        ],
        compiler_params=pltpu.CompilerParams(use_tc_tiling_on_sc=False),
    )


def _sigmoid(x):
    return 1.0 / (1.0 + jnp.exp(-x))


def _tc_body(urows_ref, irows_ref, th_ref, ph_ref, dens_ref,
             w1_ref, b1_ref, w2_ref, b2_ref, w3_ref, b3_ref,
             rho_ref, alpha_ref, theta_ref, phi_ref, pred_ref, w_scr):
    g = pl.program_id(0)

    @pl.when(g == 0)
    def _():
        a_raw = urows_ref[...]
        r_raw = irows_ref[...]
        a = jnp.exp(a_raw - jnp.max(a_raw, axis=-1, keepdims=True))
        a = a / jnp.sum(a, axis=-1, keepdims=True)
        r = jnp.exp(r_raw - jnp.max(r_raw, axis=-1, keepdims=True))
        r = r / jnp.sum(r, axis=-1, keepdims=True)
        alpha_ref[...] = a
        rho_ref[...] = r
        th = _sigmoid(th_ref[...])
        ph = _sigmoid(ph_ref[...])
        theta_ref[...] = th
        phi_ref[...] = ph
        ra = jnp.sum(r * a, axis=-1)
        w_scr[...] = (ra * th[:, 0] * ph[:, 0])[None, :]

    d = dens_ref[...]
    h = jnp.maximum(d * w1_ref[...] + b1_ref[...], 0.0)
    h = jnp.maximum(
        jnp.dot(h, w2_ref[...], preferred_element_type=jnp.float32)
        + b2_ref[...], 0.0)
    eta = jnp.dot(h, w3_ref[...], preferred_element_type=jnp.float32) \
        + b3_ref[...]
    lmbd = jnp.exp(eta)
    pred_ref[...] = lmbd * w_scr[...]


@functools.lru_cache(maxsize=None)
def _make_tc_main(B, D, RB):
    f32 = jnp.float32
    grid = (B // RB,)
    full = lambda *s: pl.BlockSpec(s, lambda g: (0,) * len(s))
    return pl.pallas_call(
        _tc_body,
        grid=grid,
        in_specs=[
            full(B, D),
            full(B, D),
            full(B, 1),
            full(B, 1),
            pl.BlockSpec((RB, 1), lambda g: (g, 0)),
            full(1, 10), full(1, 10),
            full(10, 10), full(1, 10),
            full(10, 1), full(1, 1),
        ],
        out_specs=[
            full(B, D),
            full(B, D),
            full(B, 1),
            full(B, 1),
            pl.BlockSpec((RB, B), lambda g: (g, 0)),
        ],
        out_shape=[
            jax.ShapeDtypeStruct((B, D), f32),
            jax.ShapeDtypeStruct((B, D), f32),
            jax.ShapeDtypeStruct((B, 1), f32),
            jax.ShapeDtypeStruct((B, 1), f32),
            jax.ShapeDtypeStruct((B, B), f32),
        ],
        scratch_shapes=[pltpu.VMEM((1, B), f32)],
    )


def kernel(users, items, users_degree, items_degree, density,
           emb_users, emb_items, emb_theta_users, emb_phi_items,
           logit_pi, logit_psi, W1, b1, W2, b2, W3, b3):
    B = users.shape[0]
    D = emb_users.shape[1]
    urows, irows, thg, phg = _make_sc_gather(B, D)(
        users, items, emb_users, emb_items,
        emb_theta_users.reshape(-1), emb_phi_items.reshape(-1))
    rho, alpha, theta, phi, pred = _make_tc_main(B, D, 512)(
        urows, irows, thg.reshape(B, 1), phg.reshape(B, 1), density,
        W1, b1.reshape(1, 10), W2, b2.reshape(1, 10), W3, b3.reshape(1, 1))
    empty_u = jnp.zeros((B, 0), jnp.float32)
    empty_i = jnp.zeros((B, 0), jnp.float32)
    sigma = jnp.float32(0.01)
    return (rho, alpha, empty_u, empty_i, logit_pi, logit_psi,
            theta, phi, sigma, sigma, pred)
